# baseline (speedup 1.0000x reference)
import functools

import jax
import jax.numpy as jnp
from jax import lax
from jax.experimental import pallas as pl
from jax.experimental.pallas import tpu as pltpu
from jax.experimental.pallas import tpu_sc as plsc

E = 8
K = 2
D = 1024
H = 4096
T = 4096

TM = 256
NP = T * K + E * TM
NT = NP // TM
TH = 2048

_NEG = -3.0e38


def _router_kernel(x_ref, wr_ref, ri_ref, rf_ref, xb_ref, exc_ref, ohs_ref):
    x = x_ref[...]
    xb_ref[...] = x.astype(jnp.bfloat16)
    logits = jax.lax.dot_general(
        x, wr_ref[...], (((1,), (1,)), ((), ())),
        preferred_element_type=jnp.float32)

    eidx = jax.lax.broadcasted_iota(jnp.int32, (T, E), 1)
    m1 = jnp.max(logits, axis=1, keepdims=True)
    a1 = jnp.min(jnp.where(logits == m1, eidx, E), axis=1, keepdims=True)
    masked = jnp.where(eidx == a1, _NEG, logits)
    m2 = jnp.max(masked, axis=1, keepdims=True)
    a2 = jnp.min(jnp.where(masked == m2, eidx, E), axis=1, keepdims=True)

    e2 = jnp.exp(m2 - m1)
    g1 = 1.0 / (1.0 + e2)
    g2 = e2 * g1

    p = jnp.exp(logits - m1)
    prob = p / jnp.sum(p, axis=1, keepdims=True)
    psum = jnp.sum(prob, axis=0, keepdims=True)
    oh1 = (eidx == a1).astype(jnp.float32)
    oh2 = (eidx == a2).astype(jnp.float32)
    fsum = jnp.sum(oh1 + oh2, axis=0, keepdims=True)
    aux = (E / float(T * T)) * jnp.sum(fsum * psum, axis=1, keepdims=True)

    ohs_ref[...] = oh1 + oh2
    CH = 512
    r_i = jax.lax.broadcasted_iota(jnp.int32, (CH, CH), 0)
    c_i = jax.lax.broadcasted_iota(jnp.int32, (CH, CH), 1)
    tril = (r_i >= c_i).astype(jnp.float32)

    def body(c, carry):
        blk = ohs_ref[pl.ds(c * CH, CH), :]
        inc = jax.lax.dot_general(
            tril, blk, (((1,), (0,)), ((), ())),
            preferred_element_type=jnp.float32)
        exc_ref[pl.ds(c * CH, CH), :] = carry + inc - blk
        return carry + jnp.sum(blk, axis=0, keepdims=True)

    counts = jax.lax.fori_loop(0, T // CH, body, jnp.zeros((1, E), jnp.float32))

    pc = jnp.ceil(counts / TM) * TM
    le = jax.lax.broadcasted_iota(jnp.int32, (E, E), 0)
    ge = jax.lax.broadcasted_iota(jnp.int32, (E, E), 1)
    u_strict = (le < ge).astype(jnp.float32)
    u_incl = (le <= ge).astype(jnp.float32)
    po = jax.lax.dot_general(pc, u_strict, (((1,), (0,)), ((), ())),
                             preferred_element_type=jnp.float32)
    cum = jax.lax.dot_general(pc, u_incl, (((1,), (0,)), ((), ())),
                              preferred_element_type=jnp.float32)

    exc = exc_ref[...]
    pos0 = jnp.sum(oh1 * (po + exc), axis=1, keepdims=True)
    pos1 = jnp.sum(oh2 * (po + exc), axis=1, keepdims=True)

    jt = jax.lax.broadcasted_iota(jnp.int32, (NT, E), 0).astype(jnp.float32) * TM
    te = jnp.sum((jt >= cum).astype(jnp.float32), axis=1, keepdims=True)
    te = jnp.minimum(te, float(E - 1))

    ri_ref[:, 0:1] = pos0.astype(jnp.int32)
    ri_ref[:, 1:2] = pos1.astype(jnp.int32)
    ri_ref[0:NT, 2:3] = te.astype(jnp.int32)
    rf_ref[:, 0:1] = g1
    rf_ref[:, 1:2] = g2
    rf_ref[0:8, 2:3] = jnp.broadcast_to(aux, (8, 1))


def _ffn1_kernel(te_ref, xg_ref, w1_ref, w3_ref, h_ref):
    xb = xg_ref[...]
    a = jax.lax.dot_general(xb, w1_ref[0], (((1,), (1,)), ((), ())),
                            preferred_element_type=jnp.float32)
    b = jax.lax.dot_general(xb, w3_ref[0], (((1,), (1,)), ((), ())),
                            preferred_element_type=jnp.float32)
    h_ref[...] = ((a * jax.nn.sigmoid(a)) * b).astype(jnp.bfloat16)


def _ffn2_kernel(te_ref, h_ref, w2_ref, y_ref):
    y_ref[...] = jax.lax.dot_general(
        h_ref[...], w2_ref[0], (((1,), (1,)), ((), ())),
        preferred_element_type=jnp.float32)


_SC_MESH = plsc.VectorSubcoreMesh(core_axis_name="c", subcore_axis_name="s")
_NW = 32
_CB = T // _NW


def _dispatch_kernel(x_hbm, p0_hbm, p1_hbm, xg_hbm, idx_v, rows_v, sem):
    wid = lax.axis_index("c") * 16 + lax.axis_index("s")
    SB = 64
    for k in range(K):
        p_hbm = p0_hbm if k == 0 else p1_hbm
        for s in range(_CB // SB):
            base = wid * _CB + s * SB
            pltpu.sync_copy(p_hbm.at[pl.ds(base, SB)], idx_v)
            pltpu.async_copy(x_hbm.at[pl.ds(base, SB)], rows_v, sem).wait()
            pltpu.async_copy(rows_v, xg_hbm.at[idx_v], sem).wait()


def _combine_kernel(y_hbm, p0_hbm, p1_hbm, g0_hbm, g1_hbm, out_hbm,
                    i0_v, i1_v, a_v, b_v, g0_s, g1_s, sem):
    wid = lax.axis_index("c") * 16 + lax.axis_index("s")
    SB = 32
    pltpu.sync_copy(g0_hbm.at[pl.ds(wid * _CB, _CB)], g0_s)
    pltpu.sync_copy(g1_hbm.at[pl.ds(wid * _CB, _CB)], g1_s)
    for s in range(_CB // SB):
        base = wid * _CB + s * SB
        pltpu.sync_copy(p0_hbm.at[pl.ds(base, SB)], i0_v)
        pltpu.sync_copy(p1_hbm.at[pl.ds(base, SB)], i1_v)
        pltpu.async_copy(y_hbm.at[i0_v], a_v, sem).wait()
        pltpu.async_copy(y_hbm.at[i1_v], b_v, sem).wait()

        for r16 in range(0, SB, 16):
            gv0 = g0_s[pl.ds(s * SB + r16, 16)]
            gv1 = g1_s[pl.ds(s * SB + r16, 16)]
            for rr in range(16):
                ga = gv0[rr]
                gb = gv1[rr]
                r = r16 + rr

                @pl.loop(0, D, step=16)
                def _col(c):
                    a_v[r, pl.ds(c, 16)] = (ga * a_v[r, pl.ds(c, 16)]
                                            + gb * b_v[r, pl.ds(c, 16)])

        pltpu.sync_copy(a_v, out_hbm.at[pl.ds(base, SB)])


def kernel(x, W_router, W1, W3, W2):
    B, S, _ = x.shape
    x_flat = x.reshape(T, D)

    ri, rf, xb = pl.pallas_call(
        _router_kernel,
        out_shape=(
            jax.ShapeDtypeStruct((T, E), jnp.int32),
            jax.ShapeDtypeStruct((T, E), jnp.float32),
            jax.ShapeDtypeStruct((T, D), jnp.bfloat16),
        ),
        scratch_shapes=[pltpu.VMEM((T, E), jnp.float32),
                        pltpu.VMEM((T, E), jnp.float32)],
    )(x_flat, W_router)

    pos0 = ri[:, 0]
    pos1 = ri[:, 1]
    te = ri[:NT, 2]
    g1 = rf[:, 0]
    g2 = rf[:, 1]
    aux = rf[0, 2]

    xb_i = jax.lax.bitcast_convert_type(xb.reshape(T, D // 2, 2), jnp.int32)
    xg_i = pl.kernel(
        _dispatch_kernel,
        out_type=jax.ShapeDtypeStruct((NP, D // 2), jnp.int32),
        mesh=_SC_MESH,
        scratch_types=[
            pltpu.VMEM((64,), jnp.int32),
            pltpu.VMEM((64, D // 2), jnp.int32),
            pltpu.SemaphoreType.DMA,
        ],
    )(xb_i, pos0, pos1)
    xg = jax.lax.bitcast_convert_type(xg_i, jnp.bfloat16).reshape(NP, D)

    h = pl.pallas_call(
        _ffn1_kernel,
        grid_spec=pltpu.PrefetchScalarGridSpec(
            num_scalar_prefetch=1,
            grid=(H // TH, NT),
            in_specs=[
                pl.BlockSpec((TM, D), lambda j, i, te_r: (i, 0)),
                pl.BlockSpec((1, TH, D), lambda j, i, te_r: (te_r[i], j, 0)),
                pl.BlockSpec((1, TH, D), lambda j, i, te_r: (te_r[i], j, 0)),
            ],
            out_specs=pl.BlockSpec((TM, TH), lambda j, i, te_r: (i, j)),
        ),
        out_shape=jax.ShapeDtypeStruct((NP, H), jnp.bfloat16),
    )(te, xg, W1, W3)

    y = pl.pallas_call(
        _ffn2_kernel,
        grid_spec=pltpu.PrefetchScalarGridSpec(
            num_scalar_prefetch=1,
            grid=(NT,),
            in_specs=[
                pl.BlockSpec((TM, H), lambda i, te_r: (i, 0)),
                pl.BlockSpec((1, D, H), lambda i, te_r: (te_r[i], 0, 0)),
            ],
            out_specs=pl.BlockSpec((TM, D), lambda i, te_r: (i, 0)),
        ),
        out_shape=jax.ShapeDtypeStruct((NP, D), jnp.float32),
    )(te, h, W2)

    out = pl.kernel(
        _combine_kernel,
        out_type=jax.ShapeDtypeStruct((T, D), jnp.float32),
        mesh=_SC_MESH,
        scratch_types=[
            pltpu.VMEM((32,), jnp.int32),
            pltpu.VMEM((32,), jnp.int32),
            pltpu.VMEM((32, D), jnp.float32),
            pltpu.VMEM((32, D), jnp.float32),
            pltpu.VMEM((_CB,), jnp.float32),
            pltpu.VMEM((_CB,), jnp.float32),
            pltpu.SemaphoreType.DMA,
        ],
    )(y, pos0, pos1, g1, g2)
    return out.reshape(B, S, D), aux


# speedup vs baseline: 1.5912x; 1.5912x over previous
import functools

import jax
import jax.numpy as jnp
from jax import lax
from jax.experimental import pallas as pl
from jax.experimental.pallas import tpu as pltpu
from jax.experimental.pallas import tpu_sc as plsc

E = 8
K = 2
D = 1024
H = 4096
T = 4096

TM = 256
NP = T * K + E * TM
NT = NP // TM
TH = 2048

_NEG = -3.0e38


def _router_kernel(x_ref, wr_ref, ri_ref, rf_ref, exc_ref, ohs_ref):
    x = x_ref[...]
    logits = jax.lax.dot_general(
        x, wr_ref[...], (((1,), (1,)), ((), ())),
        preferred_element_type=jnp.float32)

    eidx = jax.lax.broadcasted_iota(jnp.int32, (T, E), 1)
    m1 = jnp.max(logits, axis=1, keepdims=True)
    a1 = jnp.min(jnp.where(logits == m1, eidx, E), axis=1, keepdims=True)
    masked = jnp.where(eidx == a1, _NEG, logits)
    m2 = jnp.max(masked, axis=1, keepdims=True)
    a2 = jnp.min(jnp.where(masked == m2, eidx, E), axis=1, keepdims=True)

    e2 = jnp.exp(m2 - m1)
    g1 = 1.0 / (1.0 + e2)
    g2 = e2 * g1

    p = jnp.exp(logits - m1)
    prob = p / jnp.sum(p, axis=1, keepdims=True)
    psum = jnp.sum(prob, axis=0, keepdims=True)
    oh1 = (eidx == a1).astype(jnp.float32)
    oh2 = (eidx == a2).astype(jnp.float32)
    fsum = jnp.sum(oh1 + oh2, axis=0, keepdims=True)
    aux = (E / float(T * T)) * jnp.sum(fsum * psum, axis=1, keepdims=True)

    ohs_ref[...] = oh1 + oh2
    CH = 512
    r_i = jax.lax.broadcasted_iota(jnp.int32, (CH, CH), 0)
    c_i = jax.lax.broadcasted_iota(jnp.int32, (CH, CH), 1)
    tril = (r_i >= c_i).astype(jnp.float32)

    def body(c, carry):
        blk = ohs_ref[pl.ds(c * CH, CH), :]
        inc = jax.lax.dot_general(
            tril, blk, (((1,), (0,)), ((), ())),
            preferred_element_type=jnp.float32)
        exc_ref[pl.ds(c * CH, CH), :] = carry + inc - blk
        return carry + jnp.sum(blk, axis=0, keepdims=True)

    counts = jax.lax.fori_loop(0, T // CH, body, jnp.zeros((1, E), jnp.float32))

    pc = jnp.ceil(counts / TM) * TM
    le = jax.lax.broadcasted_iota(jnp.int32, (E, E), 0)
    ge = jax.lax.broadcasted_iota(jnp.int32, (E, E), 1)
    u_strict = (le < ge).astype(jnp.float32)
    u_incl = (le <= ge).astype(jnp.float32)
    po = jax.lax.dot_general(pc, u_strict, (((1,), (0,)), ((), ())),
                             preferred_element_type=jnp.float32)
    cum = jax.lax.dot_general(pc, u_incl, (((1,), (0,)), ((), ())),
                              preferred_element_type=jnp.float32)

    exc = exc_ref[...]
    pos0 = jnp.sum(oh1 * (po + exc), axis=1, keepdims=True)
    pos1 = jnp.sum(oh2 * (po + exc), axis=1, keepdims=True)

    jt = jax.lax.broadcasted_iota(jnp.int32, (NT, E), 0).astype(jnp.float32) * TM
    te = jnp.sum((jt >= cum).astype(jnp.float32), axis=1, keepdims=True)
    te = jnp.minimum(te, float(E - 1))

    ri_ref[:, 0:1] = pos0.astype(jnp.int32)
    ri_ref[:, 1:2] = pos1.astype(jnp.int32)
    ri_ref[0:NT, 2:3] = te.astype(jnp.int32)
    rf_ref[:, 0:1] = g1
    rf_ref[:, 1:2] = g2
    rf_ref[0:8, 2:3] = jnp.broadcast_to(aux, (8, 1))


def _ffn1_kernel(te_ref, xg_ref, w1_ref, w3_ref, h_ref):
    xb = xg_ref[...]
    a = jax.lax.dot_general(xb, w1_ref[0], (((1,), (1,)), ((), ())),
                            preferred_element_type=jnp.float32)
    b = jax.lax.dot_general(xb, w3_ref[0], (((1,), (1,)), ((), ())),
                            preferred_element_type=jnp.float32)
    h_ref[...] = ((a * jax.nn.sigmoid(a)) * b).astype(jnp.bfloat16)


def _ffn2_kernel(te_ref, h_ref, w2_ref, y_ref):
    y_ref[...] = jax.lax.dot_general(
        h_ref[...], w2_ref[0], (((1,), (1,)), ((), ())),
        preferred_element_type=jnp.float32)


_SC_MESH = plsc.VectorSubcoreMesh(core_axis_name="c", subcore_axis_name="s")
_NW = 32
_CB = T // _NW


def _dispatch_kernel(x_hbm, p0_hbm, p1_hbm, xg_hbm,
                     i0a_v, i1a_v, i0b_v, i1b_v, rows_a, rows_b, sem):
    wid = lax.axis_index("c") * 16 + lax.axis_index("s")
    SB = 32
    for half in range(2):
        ba = wid * _CB + half * 2 * SB
        bb = ba + SB
        ra = pltpu.async_copy(x_hbm.at[pl.ds(ba, SB)], rows_a, sem)
        rb = pltpu.async_copy(x_hbm.at[pl.ds(bb, SB)], rows_b, sem)
        pltpu.sync_copy(p0_hbm.at[pl.ds(ba, SB)], i0a_v)
        pltpu.sync_copy(p1_hbm.at[pl.ds(ba, SB)], i1a_v)
        pltpu.sync_copy(p0_hbm.at[pl.ds(bb, SB)], i0b_v)
        pltpu.sync_copy(p1_hbm.at[pl.ds(bb, SB)], i1b_v)
        ra.wait()
        s0 = pltpu.async_copy(rows_a, xg_hbm.at[i0a_v], sem)
        s1 = pltpu.async_copy(rows_a, xg_hbm.at[i1a_v], sem)
        rb.wait()
        s2 = pltpu.async_copy(rows_b, xg_hbm.at[i0b_v], sem)
        s3 = pltpu.async_copy(rows_b, xg_hbm.at[i1b_v], sem)
        s0.wait()
        s1.wait()
        s2.wait()
        s3.wait()


def _combine_kernel(y_hbm, p0_hbm, p1_hbm, g0_hbm, g1_hbm, out_hbm,
                    i0_v, i1_v, a_v, b_v, o_v, g0_s, g1_s, sem):
    wid = lax.axis_index("c") * 16 + lax.axis_index("s")
    SB = 32
    pltpu.sync_copy(g0_hbm.at[pl.ds(wid * _CB, _CB)], g0_s)
    pltpu.sync_copy(g1_hbm.at[pl.ds(wid * _CB, _CB)], g1_s)
    wprev = None
    for s in range(_CB // SB):
        base = wid * _CB + s * SB
        pltpu.sync_copy(p0_hbm.at[pl.ds(base, SB)], i0_v)
        pltpu.sync_copy(p1_hbm.at[pl.ds(base, SB)], i1_v)
        ca = pltpu.async_copy(y_hbm.at[i0_v], a_v, sem)
        cb = pltpu.async_copy(y_hbm.at[i1_v], b_v, sem)
        ca.wait()
        cb.wait()
        if wprev is not None:
            wprev.wait()

        for r16 in range(0, SB, 16):
            gv0 = g0_s[pl.ds(s * SB + r16, 16)]
            gv1 = g1_s[pl.ds(s * SB + r16, 16)]
            for rr in range(16):
                ga = gv0[rr]
                gb = gv1[rr]
                r = r16 + rr

                @pl.loop(0, D, step=64)
                def _col(c):
                    for u in range(4):
                        cs = pl.ds(c + u * 16, 16)
                        o_v[r, cs] = ga * a_v[r, cs] + gb * b_v[r, cs]

        wprev = pltpu.async_copy(o_v, out_hbm.at[pl.ds(base, SB)], sem)
    wprev.wait()


def kernel(x, W_router, W1, W3, W2):
    B, S, _ = x.shape
    x_flat = x.reshape(T, D)

    ri, rf = pl.pallas_call(
        _router_kernel,
        out_shape=(
            jax.ShapeDtypeStruct((T, E), jnp.int32),
            jax.ShapeDtypeStruct((T, E), jnp.float32),
        ),
        scratch_shapes=[pltpu.VMEM((T, E), jnp.float32),
                        pltpu.VMEM((T, E), jnp.float32)],
    )(x_flat, W_router)

    pos0 = ri[:, 0]
    pos1 = ri[:, 1]
    te = ri[:NT, 2]
    g1 = rf[:, 0]
    g2 = rf[:, 1]
    aux = rf[0, 2]

    xg = pl.kernel(
        _dispatch_kernel,
        out_type=jax.ShapeDtypeStruct((NP, D), jnp.float32),
        mesh=_SC_MESH,
        scratch_types=[
            pltpu.VMEM((32,), jnp.int32),
            pltpu.VMEM((32,), jnp.int32),
            pltpu.VMEM((32,), jnp.int32),
            pltpu.VMEM((32,), jnp.int32),
            pltpu.VMEM((32, D), jnp.float32),
            pltpu.VMEM((32, D), jnp.float32),
            pltpu.SemaphoreType.DMA,
        ],
    )(x_flat, pos0, pos1)

    h = pl.pallas_call(
        _ffn1_kernel,
        grid_spec=pltpu.PrefetchScalarGridSpec(
            num_scalar_prefetch=1,
            grid=(H // TH, NT),
            in_specs=[
                pl.BlockSpec((TM, D), lambda j, i, te_r: (i, 0)),
                pl.BlockSpec((1, TH, D), lambda j, i, te_r: (te_r[i], j, 0)),
                pl.BlockSpec((1, TH, D), lambda j, i, te_r: (te_r[i], j, 0)),
            ],
            out_specs=pl.BlockSpec((TM, TH), lambda j, i, te_r: (i, j)),
        ),
        out_shape=jax.ShapeDtypeStruct((NP, H), jnp.bfloat16),
    )(te, xg, W1, W3)

    y = pl.pallas_call(
        _ffn2_kernel,
        grid_spec=pltpu.PrefetchScalarGridSpec(
            num_scalar_prefetch=1,
            grid=(NT,),
            in_specs=[
                pl.BlockSpec((TM, H), lambda i, te_r: (i, 0)),
                pl.BlockSpec((1, D, H), lambda i, te_r: (te_r[i], 0, 0)),
            ],
            out_specs=pl.BlockSpec((TM, D), lambda i, te_r: (i, 0)),
        ),
        out_shape=jax.ShapeDtypeStruct((NP, D), jnp.float32),
    )(te, h, W2)

    out = pl.kernel(
        _combine_kernel,
        out_type=jax.ShapeDtypeStruct((T, D), jnp.float32),
        mesh=_SC_MESH,
        scratch_types=[
            pltpu.VMEM((32,), jnp.int32),
            pltpu.VMEM((32,), jnp.int32),
            pltpu.VMEM((32, D), jnp.float32),
            pltpu.VMEM((32, D), jnp.float32),
            pltpu.VMEM((32, D), jnp.float32),
            pltpu.VMEM((_CB,), jnp.float32),
            pltpu.VMEM((_CB,), jnp.float32),
            pltpu.SemaphoreType.DMA,
        ],
    )(y, pos0, pos1, g1, g2)
    return out.reshape(B, S, D), aux


# speedup vs baseline: 1.6705x; 1.0498x over previous
import functools

import jax
import jax.numpy as jnp
from jax import lax
from jax.experimental import pallas as pl
from jax.experimental.pallas import tpu as pltpu
from jax.experimental.pallas import tpu_sc as plsc

E = 8
K = 2
D = 1024
H = 4096
T = 4096

TM = 256
NP = T * K + E * TM
NT = NP // TM
TH = 2048

_NEG = -3.0e38


def _router_kernel(x_ref, wr_ref, ri_ref, rf_ref, exc_ref, ohs_ref):
    x = x_ref[...]
    logits = jax.lax.dot_general(
        x, wr_ref[...], (((1,), (1,)), ((), ())),
        preferred_element_type=jnp.float32)

    eidx = jax.lax.broadcasted_iota(jnp.int32, (T, E), 1)
    m1 = jnp.max(logits, axis=1, keepdims=True)
    a1 = jnp.min(jnp.where(logits == m1, eidx, E), axis=1, keepdims=True)
    masked = jnp.where(eidx == a1, _NEG, logits)
    m2 = jnp.max(masked, axis=1, keepdims=True)
    a2 = jnp.min(jnp.where(masked == m2, eidx, E), axis=1, keepdims=True)

    e2 = jnp.exp(m2 - m1)
    g1 = 1.0 / (1.0 + e2)
    g2 = e2 * g1

    p = jnp.exp(logits - m1)
    prob = p / jnp.sum(p, axis=1, keepdims=True)
    psum = jnp.sum(prob, axis=0, keepdims=True)
    oh1 = (eidx == a1).astype(jnp.float32)
    oh2 = (eidx == a2).astype(jnp.float32)
    fsum = jnp.sum(oh1 + oh2, axis=0, keepdims=True)
    aux = (E / float(T * T)) * jnp.sum(fsum * psum, axis=1, keepdims=True)

    ohs_ref[...] = oh1 + oh2
    CH = 512
    r_i = jax.lax.broadcasted_iota(jnp.int32, (CH, CH), 0)
    c_i = jax.lax.broadcasted_iota(jnp.int32, (CH, CH), 1)
    tril = (r_i >= c_i).astype(jnp.float32)

    def body(c, carry):
        blk = ohs_ref[pl.ds(c * CH, CH), :]
        inc = jax.lax.dot_general(
            tril, blk, (((1,), (0,)), ((), ())),
            preferred_element_type=jnp.float32)
        exc_ref[pl.ds(c * CH, CH), :] = carry + inc - blk
        return carry + jnp.sum(blk, axis=0, keepdims=True)

    counts = jax.lax.fori_loop(0, T // CH, body, jnp.zeros((1, E), jnp.float32))

    pc = jnp.ceil(counts / TM) * TM
    le = jax.lax.broadcasted_iota(jnp.int32, (E, E), 0)
    ge = jax.lax.broadcasted_iota(jnp.int32, (E, E), 1)
    u_strict = (le < ge).astype(jnp.float32)
    u_incl = (le <= ge).astype(jnp.float32)
    po = jax.lax.dot_general(pc, u_strict, (((1,), (0,)), ((), ())),
                             preferred_element_type=jnp.float32)
    cum = jax.lax.dot_general(pc, u_incl, (((1,), (0,)), ((), ())),
                              preferred_element_type=jnp.float32)

    exc = exc_ref[...]
    pos0 = jnp.sum(oh1 * (po + exc), axis=1, keepdims=True)
    pos1 = jnp.sum(oh2 * (po + exc), axis=1, keepdims=True)

    jt = jax.lax.broadcasted_iota(jnp.int32, (NT, E), 0).astype(jnp.float32) * TM
    te = jnp.sum((jt >= cum).astype(jnp.float32), axis=1, keepdims=True)
    te = jnp.minimum(te, float(E - 1))

    ri_ref[:, 0:1] = pos0.astype(jnp.int32)
    ri_ref[:, 1:2] = pos1.astype(jnp.int32)
    ri_ref[0:NT, 2:3] = te.astype(jnp.int32)
    rf_ref[:, 0:1] = g1
    rf_ref[:, 1:2] = g2
    rf_ref[0:8, 2:3] = jnp.broadcast_to(aux, (8, 1))


def _ffn1_kernel(te_ref, xg_ref, w1_ref, w3_ref, h_ref):
    xb = xg_ref[...]
    a = jax.lax.dot_general(xb, w1_ref[0], (((1,), (1,)), ((), ())),
                            preferred_element_type=jnp.float32)
    b = jax.lax.dot_general(xb, w3_ref[0], (((1,), (1,)), ((), ())),
                            preferred_element_type=jnp.float32)
    h_ref[...] = ((a * jax.nn.sigmoid(a)) * b).astype(jnp.bfloat16)


def _ffn2_kernel(te_ref, h_ref, w2_ref, gs_ref, y_ref):
    y = jax.lax.dot_general(
        h_ref[...], w2_ref[0], (((1,), (1,)), ((), ())),
        preferred_element_type=jnp.float32)
    y_ref[...] = y * gs_ref[:, 0:1]


_SC_MESH = plsc.VectorSubcoreMesh(core_axis_name="c", subcore_axis_name="s")
_NW = 32
_CB = T // _NW


def _dispatch_kernel(x_hbm, p0_hbm, p1_hbm, g0_hbm, g1_hbm, xg_hbm, gs_hbm,
                     i0a_v, i1a_v, i0b_v, i1b_v, rows_a, rows_b,
                     gb0_v, gb1_v, g0_s, g1_s, sem):
    wid = lax.axis_index("c") * 16 + lax.axis_index("s")
    SB = 32
    pltpu.sync_copy(g0_hbm.at[pl.ds(wid * _CB, _CB)], g0_s)
    pltpu.sync_copy(g1_hbm.at[pl.ds(wid * _CB, _CB)], g1_s)
    for half in range(2):
        ba = wid * _CB + half * 2 * SB
        bb = ba + SB
        ra = pltpu.async_copy(x_hbm.at[pl.ds(ba, SB)], rows_a, sem)
        rb = pltpu.async_copy(x_hbm.at[pl.ds(bb, SB)], rows_b, sem)
        pltpu.sync_copy(p0_hbm.at[pl.ds(ba, SB)], i0a_v)
        pltpu.sync_copy(p1_hbm.at[pl.ds(ba, SB)], i1a_v)
        pltpu.sync_copy(p0_hbm.at[pl.ds(bb, SB)], i0b_v)
        pltpu.sync_copy(p1_hbm.at[pl.ds(bb, SB)], i1b_v)
        ra.wait()
        s0 = pltpu.async_copy(rows_a, xg_hbm.at[i0a_v], sem)
        s1 = pltpu.async_copy(rows_a, xg_hbm.at[i1a_v], sem)
        rb.wait()
        s2 = pltpu.async_copy(rows_b, xg_hbm.at[i0b_v], sem)
        s3 = pltpu.async_copy(rows_b, xg_hbm.at[i1b_v], sem)
        for sub, g_s, gb_v in ((0, g0_s, gb0_v), (1, g1_s, gb1_v)):
            for r16 in range(0, 2 * SB, 16):
                gv = g_s[pl.ds(half * 2 * SB + r16, 16)]
                for rr in range(16):
                    gb_v[r16 + rr, pl.ds(0, 16)] = jnp.broadcast_to(
                        gv[rr], (16,))
        s0.wait()
        s1.wait()
        s2.wait()
        s3.wait()
        g0c = pltpu.async_copy(gb0_v.at[pl.ds(0, SB)], gs_hbm.at[i0a_v], sem)
        g1c = pltpu.async_copy(gb1_v.at[pl.ds(0, SB)], gs_hbm.at[i1a_v], sem)
        g2c = pltpu.async_copy(gb0_v.at[pl.ds(SB, SB)], gs_hbm.at[i0b_v], sem)
        g3c = pltpu.async_copy(gb1_v.at[pl.ds(SB, SB)], gs_hbm.at[i1b_v], sem)
        g0c.wait()
        g1c.wait()
        g2c.wait()
        g3c.wait()


def _combine_kernel(y_hbm, p0_hbm, p1_hbm, out_hbm,
                    i0_v, i1_v, a_v, b_v, o_v, sem):
    wid = lax.axis_index("c") * 16 + lax.axis_index("s")
    SB = 32
    wprev = None
    for s in range(_CB // SB):
        base = wid * _CB + s * SB
        pltpu.sync_copy(p0_hbm.at[pl.ds(base, SB)], i0_v)
        pltpu.sync_copy(p1_hbm.at[pl.ds(base, SB)], i1_v)
        ca = pltpu.async_copy(y_hbm.at[i0_v], a_v, sem)
        cb = pltpu.async_copy(y_hbm.at[i1_v], b_v, sem)
        ca.wait()
        cb.wait()
        if wprev is not None:
            wprev.wait()

        @pl.loop(0, SB)
        def _row(r):
            @pl.loop(0, D, step=64)
            def _col(c):
                for u in range(4):
                    cs = pl.ds(c + u * 16, 16)
                    o_v[r, cs] = a_v[r, cs] + b_v[r, cs]

        wprev = pltpu.async_copy(o_v, out_hbm.at[pl.ds(base, SB)], sem)
    wprev.wait()


def kernel(x, W_router, W1, W3, W2):
    B, S, _ = x.shape
    x_flat = x.reshape(T, D)

    ri, rf = pl.pallas_call(
        _router_kernel,
        out_shape=(
            jax.ShapeDtypeStruct((T, E), jnp.int32),
            jax.ShapeDtypeStruct((T, E), jnp.float32),
        ),
        scratch_shapes=[pltpu.VMEM((T, E), jnp.float32),
                        pltpu.VMEM((T, E), jnp.float32)],
    )(x_flat, W_router)

    pos0 = ri[:, 0]
    pos1 = ri[:, 1]
    te = ri[:NT, 2]
    g1 = rf[:, 0]
    g2 = rf[:, 1]
    aux = rf[0, 2]

    xg, gs = pl.kernel(
        _dispatch_kernel,
        out_type=(jax.ShapeDtypeStruct((NP, D), jnp.float32),
                  jax.ShapeDtypeStruct((NP, 128), jnp.float32)),
        mesh=_SC_MESH,
        scratch_types=[
            pltpu.VMEM((32,), jnp.int32),
            pltpu.VMEM((32,), jnp.int32),
            pltpu.VMEM((32,), jnp.int32),
            pltpu.VMEM((32,), jnp.int32),
            pltpu.VMEM((32, D), jnp.float32),
            pltpu.VMEM((32, D), jnp.float32),
            pltpu.VMEM((64, 128), jnp.float32),
            pltpu.VMEM((64, 128), jnp.float32),
            pltpu.VMEM((_CB,), jnp.float32),
            pltpu.VMEM((_CB,), jnp.float32),
            pltpu.SemaphoreType.DMA,
        ],
    )(x_flat, pos0, pos1, g1, g2)

    h = pl.pallas_call(
        _ffn1_kernel,
        grid_spec=pltpu.PrefetchScalarGridSpec(
            num_scalar_prefetch=1,
            grid=(H // TH, NT),
            in_specs=[
                pl.BlockSpec((TM, D), lambda j, i, te_r: (i, 0)),
                pl.BlockSpec((1, TH, D), lambda j, i, te_r: (te_r[i], j, 0)),
                pl.BlockSpec((1, TH, D), lambda j, i, te_r: (te_r[i], j, 0)),
            ],
            out_specs=pl.BlockSpec((TM, TH), lambda j, i, te_r: (i, j)),
        ),
        out_shape=jax.ShapeDtypeStruct((NP, H), jnp.bfloat16),
    )(te, xg, W1, W3)

    y = pl.pallas_call(
        _ffn2_kernel,
        grid_spec=pltpu.PrefetchScalarGridSpec(
            num_scalar_prefetch=1,
            grid=(NT,),
            in_specs=[
                pl.BlockSpec((TM, H), lambda i, te_r: (i, 0)),
                pl.BlockSpec((1, D, H), lambda i, te_r: (te_r[i], 0, 0)),
                pl.BlockSpec((TM, 128), lambda i, te_r: (i, 0)),
            ],
            out_specs=pl.BlockSpec((TM, D), lambda i, te_r: (i, 0)),
        ),
        out_shape=jax.ShapeDtypeStruct((NP, D), jnp.float32),
    )(te, h, W2, gs)

    out = pl.kernel(
        _combine_kernel,
        out_type=jax.ShapeDtypeStruct((T, D), jnp.float32),
        mesh=_SC_MESH,
        scratch_types=[
            pltpu.VMEM((32,), jnp.int32),
            pltpu.VMEM((32,), jnp.int32),
            pltpu.VMEM((32, D), jnp.float32),
            pltpu.VMEM((32, D), jnp.float32),
            pltpu.VMEM((32, D), jnp.float32),
            pltpu.SemaphoreType.DMA,
        ],
    )(y, pos0, pos1)
    return out.reshape(B, S, D), aux


# speedup vs baseline: 1.6949x; 1.0146x over previous
import functools

import jax
import jax.numpy as jnp
from jax import lax
from jax.experimental import pallas as pl
from jax.experimental.pallas import tpu as pltpu
from jax.experimental.pallas import tpu_sc as plsc

E = 8
K = 2
D = 1024
H = 4096
T = 4096

TM = 256
NP = T * K + E * TM
NT = NP // TM
TH = 2048

_NEG = -3.0e38


def _router_kernel(x_ref, wr_ref, ri_ref, rf_ref, xb_ref, exc_ref, ohs_ref):
    x = x_ref[...]
    logits = jax.lax.dot_general(
        x, wr_ref[...], (((1,), (1,)), ((), ())),
        preferred_element_type=jnp.float32)

    eidx = jax.lax.broadcasted_iota(jnp.int32, (T, E), 1)
    m1 = jnp.max(logits, axis=1, keepdims=True)
    a1 = jnp.min(jnp.where(logits == m1, eidx, E), axis=1, keepdims=True)
    masked = jnp.where(eidx == a1, _NEG, logits)
    m2 = jnp.max(masked, axis=1, keepdims=True)
    a2 = jnp.min(jnp.where(masked == m2, eidx, E), axis=1, keepdims=True)

    e2 = jnp.exp(m2 - m1)
    g1 = 1.0 / (1.0 + e2)
    g2 = e2 * g1

    p = jnp.exp(logits - m1)
    prob = p / jnp.sum(p, axis=1, keepdims=True)
    psum = jnp.sum(prob, axis=0, keepdims=True)
    oh1 = (eidx == a1).astype(jnp.float32)
    oh2 = (eidx == a2).astype(jnp.float32)
    fsum = jnp.sum(oh1 + oh2, axis=0, keepdims=True)
    aux = (E / float(T * T)) * jnp.sum(fsum * psum, axis=1, keepdims=True)

    ohs_ref[...] = oh1 + oh2
    CH = 512
    r_i = jax.lax.broadcasted_iota(jnp.int32, (CH, CH), 0)
    c_i = jax.lax.broadcasted_iota(jnp.int32, (CH, CH), 1)
    tril = (r_i >= c_i).astype(jnp.float32)

    def body(c, carry):
        blk = ohs_ref[pl.ds(c * CH, CH), :]
        inc = jax.lax.dot_general(
            tril, blk, (((1,), (0,)), ((), ())),
            preferred_element_type=jnp.float32)
        exc_ref[pl.ds(c * CH, CH), :] = carry + inc - blk
        return carry + jnp.sum(blk, axis=0, keepdims=True)

    counts = jax.lax.fori_loop(0, T // CH, body, jnp.zeros((1, E), jnp.float32))

    pc = jnp.ceil(counts / TM) * TM
    le = jax.lax.broadcasted_iota(jnp.int32, (E, E), 0)
    ge = jax.lax.broadcasted_iota(jnp.int32, (E, E), 1)
    u_strict = (le < ge).astype(jnp.float32)
    u_incl = (le <= ge).astype(jnp.float32)
    po = jax.lax.dot_general(pc, u_strict, (((1,), (0,)), ((), ())),
                             preferred_element_type=jnp.float32)
    cum = jax.lax.dot_general(pc, u_incl, (((1,), (0,)), ((), ())),
                              preferred_element_type=jnp.float32)

    exc = exc_ref[...]
    pos0 = jnp.sum(oh1 * (po + exc), axis=1, keepdims=True)
    pos1 = jnp.sum(oh2 * (po + exc), axis=1, keepdims=True)

    jt = jax.lax.broadcasted_iota(jnp.int32, (NT, E), 0).astype(jnp.float32) * TM
    te = jnp.sum((jt >= cum).astype(jnp.float32), axis=1, keepdims=True)
    te = jnp.minimum(te, float(E - 1))

    PCH = 512
    for c in range(0, T, PCH):
        xr = x_ref[c:c + PCH, :].astype(jnp.bfloat16).astype(jnp.float32)
        blo = pltpu.bitcast(xr[:, :D // 2], jnp.int32)
        bhi = pltpu.bitcast(xr[:, D // 2:], jnp.int32)
        xb_ref[c:c + PCH, :] = jax.lax.shift_right_logical(blo, 16) | (
            bhi & jnp.int32(-65536))
    ri_ref[:, 0:1] = pos0.astype(jnp.int32)
    ri_ref[:, 1:2] = pos1.astype(jnp.int32)
    ri_ref[0:NT, 2:3] = te.astype(jnp.int32)
    rf_ref[:, 0:1] = g1
    rf_ref[:, 1:2] = g2
    rf_ref[0:8, 2:3] = jnp.broadcast_to(aux, (8, 1))


def _ffn1_kernel(te_ref, xg_ref, w1_ref, w3_ref, h_ref):
    xi = xg_ref[...]
    xlo = pltpu.bitcast(jax.lax.shift_left(xi, 16), jnp.float32)
    xhi = pltpu.bitcast(xi & jnp.int32(-65536), jnp.float32)
    w1 = w1_ref[0]
    w3 = w3_ref[0]
    dn = (((1,), (1,)), ((), ()))
    a = (jax.lax.dot_general(xlo, w1[:, :D // 2], dn,
                             preferred_element_type=jnp.float32)
         + jax.lax.dot_general(xhi, w1[:, D // 2:], dn,
                               preferred_element_type=jnp.float32))
    b = (jax.lax.dot_general(xlo, w3[:, :D // 2], dn,
                             preferred_element_type=jnp.float32)
         + jax.lax.dot_general(xhi, w3[:, D // 2:], dn,
                               preferred_element_type=jnp.float32))
    h_ref[...] = ((a * jax.nn.sigmoid(a)) * b).astype(jnp.bfloat16)


def _ffn2_kernel(te_ref, h_ref, w2_ref, gs_ref, y_ref):
    y = jax.lax.dot_general(
        h_ref[...], w2_ref[0], (((1,), (1,)), ((), ())),
        preferred_element_type=jnp.float32)
    y_ref[...] = y * gs_ref[:, 0:1]


_SC_MESH = plsc.VectorSubcoreMesh(core_axis_name="c", subcore_axis_name="s")
_NW = 32
_CB = T // _NW


def _dispatch_kernel(x_hbm, p0_hbm, p1_hbm, g0_hbm, g1_hbm, xg_hbm, gs_hbm,
                     i0a_v, i1a_v, i0b_v, i1b_v, rows_a, rows_b,
                     gb0_v, gb1_v, g0_s, g1_s, sem):
    wid = lax.axis_index("c") * 16 + lax.axis_index("s")
    SB = 32
    pltpu.sync_copy(g0_hbm.at[pl.ds(wid * _CB, _CB)], g0_s)
    pltpu.sync_copy(g1_hbm.at[pl.ds(wid * _CB, _CB)], g1_s)
    for half in range(2):
        ba = wid * _CB + half * 2 * SB
        bb = ba + SB
        ra = pltpu.async_copy(x_hbm.at[pl.ds(ba, SB)], rows_a, sem)
        rb = pltpu.async_copy(x_hbm.at[pl.ds(bb, SB)], rows_b, sem)
        pltpu.sync_copy(p0_hbm.at[pl.ds(ba, SB)], i0a_v)
        pltpu.sync_copy(p1_hbm.at[pl.ds(ba, SB)], i1a_v)
        pltpu.sync_copy(p0_hbm.at[pl.ds(bb, SB)], i0b_v)
        pltpu.sync_copy(p1_hbm.at[pl.ds(bb, SB)], i1b_v)
        ra.wait()
        s0 = pltpu.async_copy(rows_a, xg_hbm.at[i0a_v], sem)
        s1 = pltpu.async_copy(rows_a, xg_hbm.at[i1a_v], sem)
        rb.wait()
        s2 = pltpu.async_copy(rows_b, xg_hbm.at[i0b_v], sem)
        s3 = pltpu.async_copy(rows_b, xg_hbm.at[i1b_v], sem)
        for sub, g_s, gb_v in ((0, g0_s, gb0_v), (1, g1_s, gb1_v)):
            for r16 in range(0, 2 * SB, 16):
                gv = g_s[pl.ds(half * 2 * SB + r16, 16)]
                for rr in range(16):
                    gb_v[r16 + rr, pl.ds(0, 16)] = jnp.broadcast_to(
                        gv[rr], (16,))
        s0.wait()
        s1.wait()
        s2.wait()
        s3.wait()
        g0c = pltpu.async_copy(gb0_v.at[pl.ds(0, SB)], gs_hbm.at[i0a_v], sem)
        g1c = pltpu.async_copy(gb1_v.at[pl.ds(0, SB)], gs_hbm.at[i1a_v], sem)
        g2c = pltpu.async_copy(gb0_v.at[pl.ds(SB, SB)], gs_hbm.at[i0b_v], sem)
        g3c = pltpu.async_copy(gb1_v.at[pl.ds(SB, SB)], gs_hbm.at[i1b_v], sem)
        g0c.wait()
        g1c.wait()
        g2c.wait()
        g3c.wait()


def _combine_kernel(y_hbm, p0_hbm, p1_hbm, out_hbm,
                    i0_v, i1_v, a_v, b_v, o_v, sem):
    wid = lax.axis_index("c") * 16 + lax.axis_index("s")
    SB = 32
    wprev = None
    for s in range(_CB // SB):
        base = wid * _CB + s * SB
        pltpu.sync_copy(p0_hbm.at[pl.ds(base, SB)], i0_v)
        pltpu.sync_copy(p1_hbm.at[pl.ds(base, SB)], i1_v)
        ca = pltpu.async_copy(y_hbm.at[i0_v], a_v, sem)
        cb = pltpu.async_copy(y_hbm.at[i1_v], b_v, sem)
        ca.wait()
        cb.wait()
        if wprev is not None:
            wprev.wait()

        @pl.loop(0, SB)
        def _row(r):
            @pl.loop(0, D, step=64)
            def _col(c):
                for u in range(4):
                    cs = pl.ds(c + u * 16, 16)
                    o_v[r, cs] = a_v[r, cs] + b_v[r, cs]

        wprev = pltpu.async_copy(o_v, out_hbm.at[pl.ds(base, SB)], sem)
    wprev.wait()


def kernel(x, W_router, W1, W3, W2):
    B, S, _ = x.shape
    x_flat = x.reshape(T, D)

    ri, rf, xb_i = pl.pallas_call(
        _router_kernel,
        out_shape=(
            jax.ShapeDtypeStruct((T, E), jnp.int32),
            jax.ShapeDtypeStruct((T, E), jnp.float32),
            jax.ShapeDtypeStruct((T, D // 2), jnp.int32),
        ),
        scratch_shapes=[pltpu.VMEM((T, E), jnp.float32),
                        pltpu.VMEM((T, E), jnp.float32)],
    )(x_flat, W_router)

    pos0 = ri[:, 0]
    pos1 = ri[:, 1]
    te = ri[:NT, 2]
    g1 = rf[:, 0]
    g2 = rf[:, 1]
    aux = rf[0, 2]

    xg_i, gs = pl.kernel(
        _dispatch_kernel,
        out_type=(jax.ShapeDtypeStruct((NP, D // 2), jnp.int32),
                  jax.ShapeDtypeStruct((NP, 128), jnp.float32)),
        mesh=_SC_MESH,
        scratch_types=[
            pltpu.VMEM((32,), jnp.int32),
            pltpu.VMEM((32,), jnp.int32),
            pltpu.VMEM((32,), jnp.int32),
            pltpu.VMEM((32,), jnp.int32),
            pltpu.VMEM((32, D // 2), jnp.int32),
            pltpu.VMEM((32, D // 2), jnp.int32),
            pltpu.VMEM((64, 128), jnp.float32),
            pltpu.VMEM((64, 128), jnp.float32),
            pltpu.VMEM((_CB,), jnp.float32),
            pltpu.VMEM((_CB,), jnp.float32),
            pltpu.SemaphoreType.DMA,
        ],
    )(xb_i, pos0, pos1, g1, g2)

    h = pl.pallas_call(
        _ffn1_kernel,
        grid_spec=pltpu.PrefetchScalarGridSpec(
            num_scalar_prefetch=1,
            grid=(H // TH, NT),
            in_specs=[
                pl.BlockSpec((TM, D // 2), lambda j, i, te_r: (i, 0)),
                pl.BlockSpec((1, TH, D), lambda j, i, te_r: (te_r[i], j, 0)),
                pl.BlockSpec((1, TH, D), lambda j, i, te_r: (te_r[i], j, 0)),
            ],
            out_specs=pl.BlockSpec((TM, TH), lambda j, i, te_r: (i, j)),
        ),
        out_shape=jax.ShapeDtypeStruct((NP, H), jnp.bfloat16),
    )(te, xg_i, W1, W3)

    y = pl.pallas_call(
        _ffn2_kernel,
        grid_spec=pltpu.PrefetchScalarGridSpec(
            num_scalar_prefetch=1,
            grid=(NT,),
            in_specs=[
                pl.BlockSpec((TM, H), lambda i, te_r: (i, 0)),
                pl.BlockSpec((1, D, H), lambda i, te_r: (te_r[i], 0, 0)),
                pl.BlockSpec((TM, 128), lambda i, te_r: (i, 0)),
            ],
            out_specs=pl.BlockSpec((TM, D), lambda i, te_r: (i, 0)),
        ),
        out_shape=jax.ShapeDtypeStruct((NP, D), jnp.float32),
    )(te, h, W2, gs)

    out = pl.kernel(
        _combine_kernel,
        out_type=jax.ShapeDtypeStruct((T, D), jnp.float32),
        mesh=_SC_MESH,
        scratch_types=[
            pltpu.VMEM((32,), jnp.int32),
            pltpu.VMEM((32,), jnp.int32),
            pltpu.VMEM((32, D), jnp.float32),
            pltpu.VMEM((32, D), jnp.float32),
            pltpu.VMEM((32, D), jnp.float32),
            pltpu.SemaphoreType.DMA,
        ],
    )(y, pos0, pos1)
    return out.reshape(B, S, D), aux
